# baseline (speedup 1.0000x reference)
import functools

import jax
import jax.numpy as jnp
from jax import lax
from jax.experimental import pallas as pl
from jax.experimental.pallas import tpu as pltpu
from jax.experimental.pallas import tpu_sc as plsc

U = 25000
I = 25000
D = 128
H = D // 2
NNZ = 400000
NC = 2
NS = 16
K = 128
CW = 1
GRP = 12
NSLOT = 4
GB = 3
CPT = -(-NNZ // (NS * K * CW * GRP)) * GRP
NNZ_PAD = CPT * NS * K * CW
RP = -(-U // (NS * 8)) * (NS * 8)
RPT = RP // NS


def _ids():
    return lax.axis_index("c"), lax.axis_index("s")


def _spmm_body(idx_h, tbl_h, zero_h, out_h,
               acc, ibuf, g,
               si0, si1, si2, si3, sg0, sg1, sg2, ss0, ss1, ss2, sz):
    cid, sid = _ids()
    sem_i = (si0, si1, si2, si3)
    sem_g = (sg0, sg1, sg2)
    sem_s = (ss0, ss1, ss2)
    base = sid * CPT

    def issue_idx(c, s):
        pltpu.async_copy(idx_h.at[base + c], ibuf.at[pl.ds(s * 3, 3)],
                         sem_i[s])

    def wait_idx(s):
        pltpu.make_async_copy(idx_h.at[0], ibuf.at[pl.ds(s * 3, 3)],
                              sem_i[s]).wait()

    def transform(s):
        @pl.loop(0, K // 16)
        def _ix(v):
            sl = pl.ds(v * 16, 16)
            ibuf[s * 3 + 1, sl] = ibuf[s * 3 + 1, sl] + cid

    def issue_gather(s, b):
        pltpu.async_copy(tbl_h.at[ibuf.at[s * 3 + 1]], g.at[b], sem_g[b])

    def wait_gather(b):
        pltpu.make_async_copy(tbl_h.at[ibuf.at[1]], g.at[b], sem_g[b]).wait()

    def issue_scatter(s, b):
        pltpu.async_copy(g.at[b], acc.at[ibuf.at[s * 3]], sem_s[b], add=True)

    def wait_scatter(b):
        pltpu.make_async_copy(g.at[b], acc.at[ibuf.at[0]], sem_s[b]).wait()

    def scale(s, b):
        @plsc.parallel_loop(0, K // 16)
        def _scale(q):
            vv = plsc.bitcast(ibuf[s * 3 + 2, pl.ds(q * 16, 16)], jnp.float32)
            for l in range(16):
                e = q * 16 + l
                vs = jnp.full((16,), vv[l])
                for f in range(H // 16):
                    sl = pl.ds(16 * f, 16)
                    g[b, e, sl] = g[b, e, sl] * vs

    zero_cp = pltpu.async_copy(zero_h, acc.at[pl.ds(sid * RPT, RPT)], sz)
    issue_idx(0, 0)
    issue_idx(1, 1)
    wait_idx(0)
    transform(0)
    zero_cp.wait()
    plsc.subcore_barrier()
    issue_gather(0, 0)

    @pl.loop(0, CPT // GRP)
    def _grp(gi):
        c0 = gi * GRP
        for j in range(GRP):
            c = c0 + j
            s = j % NSLOT
            b = j % GB

            @pl.when(c > 1)
            def _():
                wait_scatter((j - 2) % GB)

            @pl.when(c + 2 < CPT)
            def _():
                issue_idx(c + 2, (j + 2) % NSLOT)

            @pl.when(c + 1 < CPT)
            def _():
                wait_idx((j + 1) % NSLOT)
                transform((j + 1) % NSLOT)
                issue_gather((j + 1) % NSLOT, (j + 1) % GB)

            wait_gather(b)
            scale(s, b)
            issue_scatter(s, b)

    wait_scatter((CPT - 2) % GB)
    wait_scatter((CPT - 1) % GB)
    plsc.subcore_barrier()
    pltpu.sync_copy(acc.at[pl.ds(sid * RPT, RPT)],
                    out_h.at[cid, pl.ds(sid * RPT, RPT)])


@functools.cache
def _get_spmm():
    return pl.kernel(
        _spmm_body,
        out_type=jax.ShapeDtypeStruct((NC, RP, H), jnp.float32),
        mesh=plsc.VectorSubcoreMesh(core_axis_name="c", subcore_axis_name="s",
                                    num_cores=NC, num_subcores=NS),
        scratch_types=[
            pltpu.VMEM_SHARED((RP, H), jnp.float32),
            pltpu.VMEM((NSLOT * 3, K), jnp.int32),
            pltpu.VMEM((GB, K, H), jnp.float32),
        ] + [pltpu.SemaphoreType.DMA] * 11,
        compiler_params=pltpu.CompilerParams(use_tc_tiling_on_sc=False,
                                             needs_layout_passes=False),
    )


def _pad1(a, n=NNZ_PAD):
    return jnp.concatenate([a, jnp.zeros((n - NNZ,), a.dtype)])


def _prep_idx(dst, src, val):
    d = _pad1(dst).reshape(NS * CPT, K)
    s = _pad1(src * 2).reshape(NS * CPT, K)
    v = lax.bitcast_convert_type(_pad1(val), jnp.int32).reshape(NS * CPT, K)
    return jnp.stack([d, s, v], axis=1)


def _spmm(dst, src, val, tbl, zero_h):
    tbl_v = tbl.reshape(2 * tbl.shape[0], H)
    return _get_spmm()(_prep_idx(dst, src, val), tbl_v, zero_h)


BU = 1000
NB = U // BU


def _halves(a):
    return jnp.concatenate([a[0], a[1]], axis=-1)


def _tail_a_body(a0, a1, a2, a3, w, uo, o0, o1, o2, ss):
    i = pl.program_id(0)
    x0 = _halves(a0)
    x1 = _halves(a1)
    x2 = _halves(a2)
    x3 = _halves(a3)
    m = (x0 + x1 + x2 + x3) * 0.25
    z = jnp.dot(m, w[...], preferred_element_type=jnp.float32)
    uo[...] = 1.0 / (1.0 + jnp.exp(-z))
    o0[...] = x0
    o1[...] = x1
    o2[...] = x2
    sq = jnp.concatenate(
        [jnp.sum(x0 * x0, axis=0, keepdims=True),
         jnp.sum(x1 * x1, axis=0, keepdims=True),
         jnp.sum(x2 * x2, axis=0, keepdims=True),
         jnp.sum(x3 * x3, axis=0, keepdims=True),
         jnp.zeros((4, D), jnp.float32)], axis=0)

    @pl.when(i == 0)
    def _():
        ss[...] = jnp.zeros_like(ss)

    ss[...] += sq


def _tail_b_body(a0, a1, a2, a3, ss, on):
    f = 1.0 / jnp.maximum(jnp.sqrt(ss[...]), 1e-12)
    for b, a in enumerate((a0, a1, a2, a3)):
        on[b] = _halves(a) * f[b]


_in_spec_half = pl.BlockSpec((NC, BU, H), lambda i: (0, i, 0))
_out_full = pl.BlockSpec((BU, D), lambda i: (i, 0))

_TAIL_A = pl.pallas_call(
    _tail_a_body,
    grid=(NB,),
    in_specs=[_in_spec_half] * 4 + [pl.BlockSpec((D, D), lambda i: (0, 0))],
    out_specs=[_out_full] * 4 + [pl.BlockSpec((8, D), lambda i: (0, 0))],
    out_shape=[jax.ShapeDtypeStruct((U, D), jnp.float32)] * 4
              + [jax.ShapeDtypeStruct((8, D), jnp.float32)],
)

_TAIL_B = pl.pallas_call(
    _tail_b_body,
    grid=(NB,),
    in_specs=[_in_spec_half] * 4 + [pl.BlockSpec((8, D), lambda i: (0, 0))],
    out_specs=pl.BlockSpec((4, BU, D), lambda i: (0, i, 0)),
    out_shape=jax.ShapeDtypeStruct((4, U, D), jnp.float32),
)


def kernel(user_embedding, item_embedding, uu_embed0, ii_embed0,
           uu_embed1, ii_embed1, uu_embed2, ii_embed2,
           row0, col0, val0, row1, col1, val1,
           row2, col2, val2, row3, col3, val3,
           u_w, i_w):
    zero_h = jnp.zeros((RPT, H), jnp.float32)

    ue = [None] * 4
    ie = [None] * 4
    ue[3] = _spmm(row3, col3, val3, item_embedding, zero_h)
    ie[3] = _spmm(col3, row3, val3, user_embedding, zero_h)
    ue[2] = _spmm(row2, col2, val2, ii_embed2, zero_h)
    ie[2] = _spmm(col2, row2, val2, uu_embed2, zero_h)
    ue[1] = _spmm(row1, col1, val1, ii_embed1, zero_h)
    ie[1] = _spmm(col1, row1, val1, uu_embed1, zero_h)
    ue[0] = _spmm(row0, col0, val0, ii_embed0, zero_h)
    ie[0] = _spmm(col0, row0, val0, uu_embed0, zero_h)

    user_out, uu0, uu1, uu2, ssu = _TAIL_A(ue[0], ue[1], ue[2], ue[3], u_w)
    item_out, ii0, ii1, ii2, ssi = _TAIL_A(ie[0], ie[1], ie[2], ie[3], i_w)
    user_norm = _TAIL_B(ue[0], ue[1], ue[2], ue[3], ssu)
    item_norm = _TAIL_B(ie[0], ie[1], ie[2], ie[3], ssi)

    return (user_out, item_out, user_norm, item_norm,
            uu0, ii0, uu1, ii1, uu2, ii2)


# speedup vs baseline: 3.1833x; 3.1833x over previous
import functools

import jax
import jax.numpy as jnp
from jax import lax
from jax.experimental import pallas as pl
from jax.experimental.pallas import tpu as pltpu
from jax.experimental.pallas import tpu_sc as plsc

U = 25000
I = 25000
D = 128
H = D // 2
NNZ = 400000
NC = 2
NS = 16
K = 128
CW = 1
GRP = 4
NSLOT = 4
CPT = -(-NNZ // (NS * K * CW * GRP)) * GRP
NNZ_PAD = CPT * NS * K * CW
RP = -(-U // (NS * 8)) * (NS * 8)
RPT = RP // NS


def _ids():
    return lax.axis_index("c"), lax.axis_index("s")


def _spmm_body(idx_h, tbl_h, zero_h, out_h,
               acc, ibuf, g,
               si0, si1, si2, si3, sg0, sg1, ss0, ss1, sz):
    cid, sid = _ids()
    sem_i = (si0, si1, si2, si3)
    sem_g = (sg0, sg1)
    sem_s = (ss0, ss1)
    base = sid * CPT

    def issue_idx(c, s):
        pltpu.async_copy(idx_h.at[base + c], ibuf.at[pl.ds(s * 3, 3)],
                         sem_i[s])

    def wait_idx(s):
        pltpu.make_async_copy(idx_h.at[0], ibuf.at[pl.ds(s * 3, 3)],
                              sem_i[s]).wait()

    def transform(s):
        @pl.loop(0, K // 16)
        def _ix(v):
            sl = pl.ds(v * 16, 16)
            ibuf[s * 3 + 1, sl] = ibuf[s * 3 + 1, sl] + cid

    def issue_gather(s, b):
        pltpu.async_copy(tbl_h.at[ibuf.at[s * 3 + 1]], g.at[b], sem_g[b])

    def wait_gather(b):
        pltpu.make_async_copy(tbl_h.at[ibuf.at[1]], g.at[b], sem_g[b]).wait()

    def issue_scatter(s, b):
        pltpu.async_copy(g.at[b], acc.at[ibuf.at[s * 3]], sem_s[b], add=True)

    def wait_scatter(b):
        pltpu.make_async_copy(g.at[b], acc.at[ibuf.at[0]], sem_s[b]).wait()

    def scale(s, b):
        @plsc.parallel_loop(0, K // 16)
        def _scale(q):
            vv = plsc.bitcast(ibuf[s * 3 + 2, pl.ds(q * 16, 16)], jnp.float32)
            for l in range(16):
                e = q * 16 + l
                vs = jnp.full((16,), vv[l])
                for f in range(H // 16):
                    sl = pl.ds(16 * f, 16)
                    g[b, e, sl] = g[b, e, sl] * vs

    zero_cp = pltpu.async_copy(zero_h, acc.at[pl.ds(sid * RPT, RPT)], sz)
    issue_idx(0, 0)
    issue_idx(1, 1)
    wait_idx(0)
    transform(0)
    zero_cp.wait()
    plsc.subcore_barrier()
    issue_gather(0, 0)

    @pl.loop(0, CPT // GRP)
    def _grp(gi):
        c0 = gi * GRP
        for j in range(GRP):
            c = c0 + j
            s = j % NSLOT
            b = j % 2

            @pl.when(c > 0)
            def _():
                wait_scatter(1 - b)

            @pl.when(c + 2 < CPT)
            def _():
                issue_idx(c + 2, (j + 2) % NSLOT)

            @pl.when(c + 1 < CPT)
            def _():
                wait_idx((j + 1) % NSLOT)
                transform((j + 1) % NSLOT)
                issue_gather((j + 1) % NSLOT, 1 - b)

            wait_gather(b)
            scale(s, b)
            issue_scatter(s, b)

    wait_scatter((CPT - 1) % 2)
    plsc.subcore_barrier()
    pltpu.sync_copy(acc.at[pl.ds(sid * RPT, RPT)],
                    out_h.at[cid, pl.ds(sid * RPT, RPT)])


@functools.cache
def _get_spmm():
    return pl.kernel(
        _spmm_body,
        out_type=jax.ShapeDtypeStruct((NC, RP, H), jnp.float32),
        mesh=plsc.VectorSubcoreMesh(core_axis_name="c", subcore_axis_name="s",
                                    num_cores=NC, num_subcores=NS),
        scratch_types=[
            pltpu.VMEM_SHARED((RP, H), jnp.float32),
            pltpu.VMEM((NSLOT * 3, K), jnp.int32),
            pltpu.VMEM((2, K, H), jnp.float32),
        ] + [pltpu.SemaphoreType.DMA] * 9,
        compiler_params=pltpu.CompilerParams(use_tc_tiling_on_sc=False,
                                             needs_layout_passes=False),
    )


def _pad1(a, n=NNZ_PAD):
    return jnp.concatenate([a, jnp.zeros((n - NNZ,), a.dtype)])


def _prep_idx(dst, src, val):
    d = _pad1(dst).reshape(NS * CPT, K)
    s = _pad1(src * 2).reshape(NS * CPT, K)
    v = lax.bitcast_convert_type(_pad1(val), jnp.int32).reshape(NS * CPT, K)
    return jnp.stack([d, s, v], axis=1)


def _spmm(dst, src, val, tbl, zero_h):
    tbl_v = tbl.reshape(2 * tbl.shape[0], H)
    return _get_spmm()(_prep_idx(dst, src, val), tbl_v, zero_h)


BU = 1000
NB = U // BU


def _halves(a):
    return jnp.concatenate([a[0], a[1]], axis=-1)


def _tail_a_body(a0, a1, a2, a3, w, uo, o0, o1, o2, ss):
    i = pl.program_id(0)
    x0 = _halves(a0)
    x1 = _halves(a1)
    x2 = _halves(a2)
    x3 = _halves(a3)
    m = (x0 + x1 + x2 + x3) * 0.25
    z = jnp.dot(m, w[...], preferred_element_type=jnp.float32)
    uo[...] = 1.0 / (1.0 + jnp.exp(-z))
    o0[...] = x0
    o1[...] = x1
    o2[...] = x2
    sq = jnp.concatenate(
        [jnp.sum(x0 * x0, axis=0, keepdims=True),
         jnp.sum(x1 * x1, axis=0, keepdims=True),
         jnp.sum(x2 * x2, axis=0, keepdims=True),
         jnp.sum(x3 * x3, axis=0, keepdims=True),
         jnp.zeros((4, D), jnp.float32)], axis=0)

    @pl.when(i == 0)
    def _():
        ss[...] = jnp.zeros_like(ss)

    ss[...] += sq


def _tail_b_body(a0, a1, a2, a3, ss, on):
    f = 1.0 / jnp.maximum(jnp.sqrt(ss[...]), 1e-12)
    for b, a in enumerate((a0, a1, a2, a3)):
        on[b] = _halves(a) * f[b]


_in_spec_half = pl.BlockSpec((NC, BU, H), lambda i: (0, i, 0))
_out_full = pl.BlockSpec((BU, D), lambda i: (i, 0))

_TAIL_A = pl.pallas_call(
    _tail_a_body,
    grid=(NB,),
    in_specs=[_in_spec_half] * 4 + [pl.BlockSpec((D, D), lambda i: (0, 0))],
    out_specs=[_out_full] * 4 + [pl.BlockSpec((8, D), lambda i: (0, 0))],
    out_shape=[jax.ShapeDtypeStruct((U, D), jnp.float32)] * 4
              + [jax.ShapeDtypeStruct((8, D), jnp.float32)],
)

_TAIL_B = pl.pallas_call(
    _tail_b_body,
    grid=(NB,),
    in_specs=[_in_spec_half] * 4 + [pl.BlockSpec((8, D), lambda i: (0, 0))],
    out_specs=pl.BlockSpec((4, BU, D), lambda i: (0, i, 0)),
    out_shape=jax.ShapeDtypeStruct((4, U, D), jnp.float32),
)


def kernel(user_embedding, item_embedding, uu_embed0, ii_embed0,
           uu_embed1, ii_embed1, uu_embed2, ii_embed2,
           row0, col0, val0, row1, col1, val1,
           row2, col2, val2, row3, col3, val3,
           u_w, i_w):
    zero_h = jnp.zeros((RPT, H), jnp.float32)

    ue = [None] * 4
    ie = [None] * 4
    ue[3] = _spmm(row3, col3, val3, item_embedding, zero_h)
    ie[3] = _spmm(col3, row3, val3, user_embedding, zero_h)
    ue[2] = _spmm(row2, col2, val2, ii_embed2, zero_h)
    ie[2] = _spmm(col2, row2, val2, uu_embed2, zero_h)
    ue[1] = _spmm(row1, col1, val1, ii_embed1, zero_h)
    ie[1] = _spmm(col1, row1, val1, uu_embed1, zero_h)
    ue[0] = _spmm(row0, col0, val0, ii_embed0, zero_h)
    ie[0] = _spmm(col0, row0, val0, uu_embed0, zero_h)

    user_out, uu0, uu1, uu2, ssu = _TAIL_A(ue[0], ue[1], ue[2], ue[3], u_w)
    item_out, ii0, ii1, ii2, ssi = _TAIL_A(ie[0], ie[1], ie[2], ie[3], i_w)
    user_norm = _TAIL_B(ue[0], ue[1], ue[2], ue[3], ssu)
    item_norm = _TAIL_B(ie[0], ie[1], ie[2], ie[3], ssi)

    return (user_out, item_out, user_norm, item_norm,
            uu0, ii0, uu1, ii1, uu2, ii2)


# speedup vs baseline: 3.3222x; 1.0437x over previous
import functools

import jax
import jax.numpy as jnp
from jax import lax
from jax.experimental import pallas as pl
from jax.experimental.pallas import tpu as pltpu
from jax.experimental.pallas import tpu_sc as plsc

U = 25000
I = 25000
D = 128
H = D // 2
NNZ = 400000
NC = 2
NS = 16
K = 112
CW = 1
GRP = 4
NSLOT = 4
CPT = -(-NNZ // (NS * K * CW * GRP)) * GRP
NNZ_PAD = CPT * NS * K * CW
RP = -(-U // (NS * 8)) * (NS * 8)
RPT = RP // NS


def _ids():
    return lax.axis_index("c"), lax.axis_index("s")


def _spmm_body(idx_h, tbl_h, zero_h, out_h,
               acc, ibuf, g,
               si0, si1, si2, si3, sg0, sg1, sg2, sg3,
               ss0, ss1, ss2, ss3, sz):
    cid, sid = _ids()
    sem_i = (si0, si1, si2, si3)
    sem_g = (sg0, sg1, sg2, sg3)
    sem_s = (ss0, ss1, ss2, ss3)
    base = sid * CPT

    def issue_idx(c, s):
        pltpu.async_copy(idx_h.at[base + c], ibuf.at[pl.ds(s * 3, 3)],
                         sem_i[s])

    def wait_idx(s):
        pltpu.make_async_copy(idx_h.at[0], ibuf.at[pl.ds(s * 3, 3)],
                              sem_i[s]).wait()

    def transform(s):
        @pl.loop(0, K // 16)
        def _ix(v):
            sl = pl.ds(v * 16, 16)
            ibuf[s * 3 + 1, sl] = ibuf[s * 3 + 1, sl] + cid

    def issue_gather(s, b):
        pltpu.async_copy(tbl_h.at[ibuf.at[s * 3 + 1]], g.at[b], sem_g[b])

    def wait_gather(b):
        pltpu.make_async_copy(tbl_h.at[ibuf.at[1]], g.at[b], sem_g[b]).wait()

    def issue_scatter(s, b):
        pltpu.async_copy(g.at[b], acc.at[ibuf.at[s * 3]], sem_s[b], add=True)

    def wait_scatter(b):
        pltpu.make_async_copy(g.at[b], acc.at[ibuf.at[0]], sem_s[b]).wait()

    def scale(s, b):
        @plsc.parallel_loop(0, K // 16)
        def _scale(q):
            vv = plsc.bitcast(ibuf[s * 3 + 2, pl.ds(q * 16, 16)], jnp.float32)
            for l in range(16):
                e = q * 16 + l
                vs = jnp.full((16,), vv[l])
                for f in range(H // 16):
                    sl = pl.ds(16 * f, 16)
                    g[b, e, sl] = g[b, e, sl] * vs

    zero_cp = pltpu.async_copy(zero_h, acc.at[pl.ds(sid * RPT, RPT)], sz)
    issue_idx(0, 0)
    issue_idx(1, 1)
    wait_idx(0)
    transform(0)
    zero_cp.wait()
    plsc.subcore_barrier()
    issue_gather(0, 0)

    @pl.loop(0, CPT // GRP)
    def _grp(gi):
        c0 = gi * GRP
        for j in range(GRP):
            c = c0 + j
            s = j % NSLOT
            b = j % NSLOT

            @pl.when(c > 1)
            def _():
                wait_scatter((j + 2) % NSLOT)

            @pl.when(c + 2 < CPT)
            def _():
                issue_idx(c + 2, (j + 2) % NSLOT)

            @pl.when(c + 1 < CPT)
            def _():
                wait_idx((j + 1) % NSLOT)
                transform((j + 1) % NSLOT)
                issue_gather((j + 1) % NSLOT, (j + 1) % NSLOT)

            wait_gather(b)
            scale(s, b)
            issue_scatter(s, b)

    wait_scatter((CPT - 2) % NSLOT)
    wait_scatter((CPT - 1) % NSLOT)
    plsc.subcore_barrier()
    pltpu.sync_copy(acc.at[pl.ds(sid * RPT, RPT)],
                    out_h.at[cid, pl.ds(sid * RPT, RPT)])


@functools.cache
def _get_spmm():
    return pl.kernel(
        _spmm_body,
        out_type=jax.ShapeDtypeStruct((NC, RP, H), jnp.float32),
        mesh=plsc.VectorSubcoreMesh(core_axis_name="c", subcore_axis_name="s",
                                    num_cores=NC, num_subcores=NS),
        scratch_types=[
            pltpu.VMEM_SHARED((RP, H), jnp.float32),
            pltpu.VMEM((NSLOT * 3, K), jnp.int32),
            pltpu.VMEM((NSLOT, K, H), jnp.float32),
        ] + [pltpu.SemaphoreType.DMA] * 13,
        compiler_params=pltpu.CompilerParams(use_tc_tiling_on_sc=False,
                                             needs_layout_passes=False),
    )


def _pad1(a, n=NNZ_PAD):
    return jnp.concatenate([a, jnp.zeros((n - NNZ,), a.dtype)])


def _prep_idx(dst, src, val):
    d = _pad1(dst).reshape(NS * CPT, K)
    s = _pad1(src * 2).reshape(NS * CPT, K)
    v = lax.bitcast_convert_type(_pad1(val), jnp.int32).reshape(NS * CPT, K)
    return jnp.stack([d, s, v], axis=1)


def _spmm(dst, src, val, tbl, zero_h):
    tbl_v = tbl.reshape(2 * tbl.shape[0], H)
    return _get_spmm()(_prep_idx(dst, src, val), tbl_v, zero_h)


BU = 1000
NB = U // BU


def _halves(a):
    return jnp.concatenate([a[0], a[1]], axis=-1)


def _tail_a_body(a0, a1, a2, a3, w, uo, o0, o1, o2, ss):
    i = pl.program_id(0)
    x0 = _halves(a0)
    x1 = _halves(a1)
    x2 = _halves(a2)
    x3 = _halves(a3)
    m = (x0 + x1 + x2 + x3) * 0.25
    z = jnp.dot(m, w[...], preferred_element_type=jnp.float32)
    uo[...] = 1.0 / (1.0 + jnp.exp(-z))
    o0[...] = x0
    o1[...] = x1
    o2[...] = x2
    sq = jnp.concatenate(
        [jnp.sum(x0 * x0, axis=0, keepdims=True),
         jnp.sum(x1 * x1, axis=0, keepdims=True),
         jnp.sum(x2 * x2, axis=0, keepdims=True),
         jnp.sum(x3 * x3, axis=0, keepdims=True),
         jnp.zeros((4, D), jnp.float32)], axis=0)

    @pl.when(i == 0)
    def _():
        ss[...] = jnp.zeros_like(ss)

    ss[...] += sq


def _tail_b_body(a0, a1, a2, a3, ss, on):
    f = 1.0 / jnp.maximum(jnp.sqrt(ss[...]), 1e-12)
    for b, a in enumerate((a0, a1, a2, a3)):
        on[b] = _halves(a) * f[b]


_in_spec_half = pl.BlockSpec((NC, BU, H), lambda i: (0, i, 0))
_out_full = pl.BlockSpec((BU, D), lambda i: (i, 0))

_TAIL_A = pl.pallas_call(
    _tail_a_body,
    grid=(NB,),
    in_specs=[_in_spec_half] * 4 + [pl.BlockSpec((D, D), lambda i: (0, 0))],
    out_specs=[_out_full] * 4 + [pl.BlockSpec((8, D), lambda i: (0, 0))],
    out_shape=[jax.ShapeDtypeStruct((U, D), jnp.float32)] * 4
              + [jax.ShapeDtypeStruct((8, D), jnp.float32)],
)

_TAIL_B = pl.pallas_call(
    _tail_b_body,
    grid=(NB,),
    in_specs=[_in_spec_half] * 4 + [pl.BlockSpec((8, D), lambda i: (0, 0))],
    out_specs=pl.BlockSpec((4, BU, D), lambda i: (0, i, 0)),
    out_shape=jax.ShapeDtypeStruct((4, U, D), jnp.float32),
)


def kernel(user_embedding, item_embedding, uu_embed0, ii_embed0,
           uu_embed1, ii_embed1, uu_embed2, ii_embed2,
           row0, col0, val0, row1, col1, val1,
           row2, col2, val2, row3, col3, val3,
           u_w, i_w):
    zero_h = jnp.zeros((RPT, H), jnp.float32)

    ue = [None] * 4
    ie = [None] * 4
    ue[3] = _spmm(row3, col3, val3, item_embedding, zero_h)
    ie[3] = _spmm(col3, row3, val3, user_embedding, zero_h)
    ue[2] = _spmm(row2, col2, val2, ii_embed2, zero_h)
    ie[2] = _spmm(col2, row2, val2, uu_embed2, zero_h)
    ue[1] = _spmm(row1, col1, val1, ii_embed1, zero_h)
    ie[1] = _spmm(col1, row1, val1, uu_embed1, zero_h)
    ue[0] = _spmm(row0, col0, val0, ii_embed0, zero_h)
    ie[0] = _spmm(col0, row0, val0, uu_embed0, zero_h)

    user_out, uu0, uu1, uu2, ssu = _TAIL_A(ue[0], ue[1], ue[2], ue[3], u_w)
    item_out, ii0, ii1, ii2, ssi = _TAIL_A(ie[0], ie[1], ie[2], ie[3], i_w)
    user_norm = _TAIL_B(ue[0], ue[1], ue[2], ue[3], ssu)
    item_norm = _TAIL_B(ie[0], ie[1], ie[2], ie[3], ssi)

    return (user_out, item_out, user_norm, item_norm,
            uu0, ii0, uu1, ii1, uu2, ii2)
